# Initial kernel scaffold; baseline (speedup 1.0000x reference)
import jax
import jax.numpy as jnp
from jax.experimental import pallas as pl


def kernel(embeddings, label):
    raise NotImplementedError("write your pallas kernel here")


# speedup vs baseline: 1.7284x; 1.7284x over previous
import functools

import jax
import jax.numpy as jnp
from jax import lax
from jax.experimental import pallas as pl
from jax.experimental.pallas import tpu as pltpu
from jax.experimental.pallas import tpu_sc as plsc

_SCALE = 8.0
_MARGIN = 0.2
_BS = 1024
_NCLS = 100000

_BLK = 4096
_GRID = (_NCLS + _BLK - 1) // _BLK
_REM = _NCLS - (_GRID - 1) * _BLK


def _dense_body(e_ref, g_ref, out_ref, acc_ref):
    pid = pl.program_id(0)

    @pl.when(pid == 0)
    def _():
        acc_ref[...] = jnp.zeros_like(acc_ref)

    x = e_ref[...]
    z = jnp.exp(_SCALE / (1.0 + jnp.exp(-x)))

    @pl.when(pid < _GRID - 1)
    def _():
        acc_ref[...] += jnp.sum(z, axis=1, keepdims=True)

    @pl.when(pid == _GRID - 1)
    def _():
        col = lax.broadcasted_iota(jnp.int32, z.shape, 1)
        zm = jnp.where(col < _REM, z, 0.0)
        s_full = acc_ref[...] + jnp.sum(zm, axis=1, keepdims=True)
        g = g_ref[...]
        sig = 1.0 / (1.0 + jnp.exp(-g))
        t = _SCALE * sig - _SCALE * _MARGIN
        s_adj = s_full - jnp.exp(_SCALE * sig) + jnp.exp(t)
        per_row = jnp.log(s_adj) - t
        out_ref[...] = jnp.sum(per_row, axis=0, keepdims=True) * (1.0 / _BS)


def _dense(e, g):
    return pl.pallas_call(
        _dense_body,
        grid=(_GRID,),
        in_specs=[
            pl.BlockSpec((_BS, _BLK), lambda i: (0, i)),
            pl.BlockSpec((_BS, 1), lambda i: (0, 0)),
        ],
        out_specs=pl.BlockSpec((1, 1), lambda i: (0, 0)),
        out_shape=jax.ShapeDtypeStruct((1, 1), jnp.float32),
        scratch_shapes=[pltpu.VMEM((_BS, 1), jnp.float32)],
    )(e, g)


def _sc_gather(flat_e, label):
    info = plsc.get_sparse_core_info()
    nc, ns, nl = info.num_cores, info.num_subcores, info.num_lanes
    nw = nc * ns
    bpw = _BS // nw
    mesh = plsc.VectorSubcoreMesh(core_axis_name="c", subcore_axis_name="s")

    @functools.partial(
        pl.kernel,
        mesh=mesh,
        out_type=jax.ShapeDtypeStruct((_BS,), jnp.float32),
        scratch_types=[
            pltpu.VMEM((bpw,), jnp.int32),
            pltpu.VMEM((bpw,), jnp.float32),
            pltpu.SemaphoreType.DMA,
        ],
    )
    def k(flat_hbm, lbl_hbm, out_hbm, idx_v, vals_v, sem):
        wid = lax.axis_index("s") * nc + lax.axis_index("c")
        base = wid * bpw
        pltpu.sync_copy(lbl_hbm.at[pl.ds(base, bpw)], idx_v)
        for j in range(bpw // nl):
            lbl = idx_v[pl.ds(j * nl, nl)]
            rows = base + j * nl + lax.iota(jnp.int32, nl)
            idx_v[pl.ds(j * nl, nl)] = rows * _NCLS + lbl
        pltpu.async_copy(flat_hbm.at[idx_v], vals_v, sem).wait()
        pltpu.sync_copy(vals_v, out_hbm.at[pl.ds(base, bpw)])

    return k(flat_e, label)


def kernel(embeddings, label):
    g = _sc_gather(embeddings.reshape(-1), label.astype(jnp.int32))
    loss = _dense(embeddings, g.reshape(_BS, 1))
    return loss[0, 0]


# speedup vs baseline: 1.7300x; 1.0009x over previous
import functools

import jax
import jax.numpy as jnp
from jax import lax
from jax.experimental import pallas as pl
from jax.experimental.pallas import tpu as pltpu
from jax.experimental.pallas import tpu_sc as plsc

_SCALE = 8.0
_MARGIN = 0.2
_BS = 1024
_NCLS = 100000

_BLK = 2048
_GRID = (_NCLS + _BLK - 1) // _BLK
_REM = _NCLS - (_GRID - 1) * _BLK


def _dense_body(e_ref, g_ref, out_ref, acc_ref):
    pid = pl.program_id(0)

    @pl.when(pid == 0)
    def _():
        acc_ref[...] = jnp.zeros_like(acc_ref)

    x = e_ref[...]
    z = jnp.exp(_SCALE / (1.0 + jnp.exp(-x)))

    @pl.when(pid < _GRID - 1)
    def _():
        acc_ref[...] += jnp.sum(z, axis=1, keepdims=True)

    @pl.when(pid == _GRID - 1)
    def _():
        col = lax.broadcasted_iota(jnp.int32, z.shape, 1)
        zm = jnp.where(col < _REM, z, 0.0)
        s_full = acc_ref[...] + jnp.sum(zm, axis=1, keepdims=True)
        g = g_ref[...]
        sig = 1.0 / (1.0 + jnp.exp(-g))
        t = _SCALE * sig - _SCALE * _MARGIN
        s_adj = s_full - jnp.exp(_SCALE * sig) + jnp.exp(t)
        per_row = jnp.log(s_adj) - t
        out_ref[...] = jnp.sum(per_row, axis=0, keepdims=True) * (1.0 / _BS)


def _dense(e, g):
    return pl.pallas_call(
        _dense_body,
        grid=(_GRID,),
        in_specs=[
            pl.BlockSpec((_BS, _BLK), lambda i: (0, i)),
            pl.BlockSpec((_BS, 1), lambda i: (0, 0)),
        ],
        out_specs=pl.BlockSpec((1, 1), lambda i: (0, 0)),
        out_shape=jax.ShapeDtypeStruct((1, 1), jnp.float32),
        scratch_shapes=[pltpu.VMEM((_BS, 1), jnp.float32)],
    )(e, g)


def _sc_gather(flat_e, label):
    info = plsc.get_sparse_core_info()
    nc, ns, nl = info.num_cores, info.num_subcores, info.num_lanes
    nw = nc * ns
    bpw = _BS // nw
    mesh = plsc.VectorSubcoreMesh(core_axis_name="c", subcore_axis_name="s")

    @functools.partial(
        pl.kernel,
        mesh=mesh,
        out_type=jax.ShapeDtypeStruct((_BS,), jnp.float32),
        scratch_types=[
            pltpu.VMEM((bpw,), jnp.int32),
            pltpu.VMEM((bpw,), jnp.float32),
            pltpu.SemaphoreType.DMA,
        ],
    )
    def k(flat_hbm, lbl_hbm, out_hbm, idx_v, vals_v, sem):
        wid = lax.axis_index("s") * nc + lax.axis_index("c")
        base = wid * bpw
        pltpu.sync_copy(lbl_hbm.at[pl.ds(base, bpw)], idx_v)
        for j in range(bpw // nl):
            lbl = idx_v[pl.ds(j * nl, nl)]
            rows = base + j * nl + lax.iota(jnp.int32, nl)
            idx_v[pl.ds(j * nl, nl)] = rows * _NCLS + lbl
        pltpu.async_copy(flat_hbm.at[idx_v], vals_v, sem).wait()
        pltpu.sync_copy(vals_v, out_hbm.at[pl.ds(base, bpw)])

    return k(flat_e, label)


def kernel(embeddings, label):
    g = _sc_gather(embeddings.reshape(-1), label.astype(jnp.int32))
    loss = _dense(embeddings, g.reshape(_BS, 1))
    return loss[0, 0]


# speedup vs baseline: 1.8152x; 1.0493x over previous
import functools

import jax
import jax.numpy as jnp
from jax import lax
from jax.experimental import pallas as pl
from jax.experimental.pallas import tpu as pltpu
from jax.experimental.pallas import tpu_sc as plsc

_SCALE = 8.0
_MARGIN = 0.2
_BS = 1024
_NCLS = 100000

_RB = 32
_GRID = _BS // _RB
_C = (_SCALE / 2.0) * 1.4426950408889634
_E4 = 54.598150033144236


def _dense_body(e_ref, g_ref, out_ref, acc_ref):
    i = pl.program_id(0)

    @pl.when(i == 0)
    def _():
        acc_ref[...] = jnp.zeros_like(acc_ref)

    x = e_ref[...]
    z2 = jnp.exp2(_C * jnp.tanh(0.5 * x))
    s2 = jnp.sum(z2, axis=1, keepdims=True)
    g = g_ref[...]
    sig = 0.5 + 0.5 * jnp.tanh(0.5 * g)
    t = _SCALE * sig - _SCALE * _MARGIN
    s_adj = _E4 * s2 - jnp.exp(_SCALE * sig) + jnp.exp(t)
    per_row = jnp.log(s_adj) - t
    acc_ref[...] += jnp.sum(per_row, axis=0, keepdims=True)

    @pl.when(i == _GRID - 1)
    def _():
        out_ref[...] = acc_ref[...] * (1.0 / _BS)


def _dense(e, g):
    return pl.pallas_call(
        _dense_body,
        grid=(_GRID,),
        in_specs=[
            pl.BlockSpec((_RB, _NCLS), lambda i: (i, 0)),
            pl.BlockSpec((_RB, 1), lambda i: (i, 0)),
        ],
        out_specs=pl.BlockSpec((1, 1), lambda i: (0, 0)),
        out_shape=jax.ShapeDtypeStruct((1, 1), jnp.float32),
        scratch_shapes=[pltpu.VMEM((1, 1), jnp.float32)],
    )(e, g)


def _sc_gather(flat_e, label):
    info = plsc.get_sparse_core_info()
    nc, ns, nl = info.num_cores, info.num_subcores, info.num_lanes
    nw = nc * ns
    bpw = _BS // nw
    mesh = plsc.VectorSubcoreMesh(core_axis_name="c", subcore_axis_name="s")

    @functools.partial(
        pl.kernel,
        mesh=mesh,
        out_type=jax.ShapeDtypeStruct((_BS,), jnp.float32),
        scratch_types=[
            pltpu.VMEM((bpw,), jnp.int32),
            pltpu.VMEM((bpw,), jnp.float32),
            pltpu.SemaphoreType.DMA,
        ],
    )
    def k(flat_hbm, lbl_hbm, out_hbm, idx_v, vals_v, sem):
        wid = lax.axis_index("s") * nc + lax.axis_index("c")
        base = wid * bpw
        pltpu.sync_copy(lbl_hbm.at[pl.ds(base, bpw)], idx_v)
        for j in range(bpw // nl):
            lbl = idx_v[pl.ds(j * nl, nl)]
            rows = base + j * nl + lax.iota(jnp.int32, nl)
            idx_v[pl.ds(j * nl, nl)] = rows * _NCLS + lbl
        pltpu.async_copy(flat_hbm.at[idx_v], vals_v, sem).wait()
        pltpu.sync_copy(vals_v, out_hbm.at[pl.ds(base, bpw)])

    return k(flat_e, label)


def kernel(embeddings, label):
    g = _sc_gather(embeddings.reshape(-1), label.astype(jnp.int32))
    loss = _dense(embeddings, g.reshape(_BS, 1))
    return loss[0, 0]


# speedup vs baseline: 3.8474x; 2.1195x over previous
import functools

import jax
import jax.numpy as jnp
from jax import lax
from jax.experimental import pallas as pl
from jax.experimental.pallas import tpu as pltpu
from jax.experimental.pallas import tpu_sc as plsc

_SCALE = 8.0
_MARGIN = 0.2
_BS = 1024
_NCLS = 100000

_RB = 32
_GRID = _BS // _RB
_C = (_SCALE / 2.0) * 1.4426950408889634
_E4 = 54.598150033144236


def _dense_body(e_ref, g_ref, out_ref, acc_ref):
    i = pl.program_id(0)

    @pl.when(i == 0)
    def _():
        acc_ref[...] = jnp.zeros_like(acc_ref)

    x = e_ref[...]
    z2 = jnp.exp2(_C * jnp.tanh(0.5 * x))
    s2 = jnp.sum(z2, axis=1, keepdims=True)
    g = g_ref[...]
    sig = 0.5 + 0.5 * jnp.tanh(0.5 * g)
    t = _SCALE * sig - _SCALE * _MARGIN
    s_adj = _E4 * s2 - jnp.exp(_SCALE * sig) + jnp.exp(t)
    per_row = jnp.log(s_adj) - t
    acc_ref[...] += jnp.sum(per_row, axis=0, keepdims=True)

    @pl.when(i == _GRID - 1)
    def _():
        out_ref[...] = acc_ref[...] * (1.0 / _BS)


def _dense(e, g):
    return pl.pallas_call(
        _dense_body,
        grid=(_GRID,),
        in_specs=[
            pl.BlockSpec((_RB, _NCLS), lambda i: (i, 0)),
            pl.BlockSpec((_RB, 1), lambda i: (i, 0)),
        ],
        out_specs=pl.BlockSpec((1, 1), lambda i: (0, 0)),
        out_shape=jax.ShapeDtypeStruct((1, 1), jnp.float32),
        scratch_shapes=[pltpu.VMEM((1, 1), jnp.float32)],
    )(e, g)


def _sc_gather(e, label):
    info = plsc.get_sparse_core_info()
    nc, ns, nl = info.num_cores, info.num_subcores, info.num_lanes
    nw = nc * ns
    bpw = _BS // nw
    mesh = plsc.VectorSubcoreMesh(core_axis_name="c", subcore_axis_name="s")

    @functools.partial(
        pl.kernel,
        mesh=mesh,
        out_type=jax.ShapeDtypeStruct((_BS,), jnp.float32),
        scratch_types=[
            pltpu.VMEM((bpw,), jnp.int32),
            pltpu.VMEM((bpw, 8, 128), jnp.float32),
            pltpu.VMEM((bpw,), jnp.float32),
            pltpu.SemaphoreType.DMA,
        ],
        compiler_params=pltpu.CompilerParams(
            use_tc_tiling_on_sc=True, needs_layout_passes=False),
    )
    def k(e_hbm, lbl_hbm, out_hbm, lbl_v, win_v, out_v, sem):
        wid = lax.axis_index("s") * nc + lax.axis_index("c")
        base = wid * bpw
        pltpu.sync_copy(lbl_hbm.at[pl.ds(base, bpw)], lbl_v)
        lane = lax.iota(jnp.int32, nl)
        copies = []
        for r in range(bpw):
            lbl16 = lbl_v[pl.ds((r // nl) * nl, nl)]
            l_r = jnp.sum(jnp.where(lane == (r % nl), lbl16, 0))
            tcol = pl.multiple_of(
                jnp.left_shift(jnp.right_shift(l_r, 7), 7), 128)
            rt = base + (r // 8) * 8
            cp = pltpu.make_async_copy(
                e_hbm.at[pl.ds(rt, 8), pl.ds(tcol, 128)], win_v.at[r], sem)
            cp.start()
            copies.append(cp)
        for cp in copies:
            cp.wait()
        for j in range(bpw // nl):
            lbl = lbl_v[pl.ds(j * nl, nl)]
            ridx = j * nl + lax.iota(jnp.int32, nl)
            sub = jnp.bitwise_and(ridx, 7)
            off = jnp.bitwise_and(lbl, 127)
            out_v[pl.ds(j * nl, nl)] = plsc.load_gather(win_v, [ridx, sub, off])
        pltpu.sync_copy(out_v, out_hbm.at[pl.ds(base, bpw)])

    return k(e, label)


def kernel(embeddings, label):
    g = _sc_gather(embeddings, label.astype(jnp.int32))
    loss = _dense(embeddings, g.reshape(_BS, 1))
    return loss[0, 0]
